# Initial kernel scaffold; baseline (speedup 1.0000x reference)
import jax
import jax.numpy as jnp
from jax.experimental import pallas as pl


def kernel(x, edge_index, W1, b1, W2, b2, W3, b3, Wm1, bm1, Wm2, bm2, Wm3, bm3):
    raise NotImplementedError("write your pallas kernel here")


# speedup vs baseline: 20.1333x; 20.1333x over previous
import functools

import jax
import jax.numpy as jnp
from jax import lax
from jax.experimental import pallas as pl
from jax.experimental.pallas import tpu as pltpu
from jax.experimental.pallas import tpu_sc as plsc

N = 10000
NP = 10240
LANES = 16
NW = 32


def _leaky(v):
    return jnp.where(v > 0, v, 0.1 * v)


def _mesh():
    return plsc.VectorSubcoreMesh(core_axis_name="c", subcore_axis_name="s")


def _sc_degree(dst_pad, EP):
    EPT = EP // NW
    CH = 2048

    @functools.partial(
        pl.kernel,
        mesh=_mesh(),
        compiler_params=pltpu.CompilerParams(needs_layout_passes=False),
        out_type=jax.ShapeDtypeStruct((NW, NP), jnp.float32),
        scratch_types=[
            pltpu.VMEM((CH,), jnp.int32),
            pltpu.VMEM((NP,), jnp.float32),
        ],
    )
    def k(dst_hbm, out_hbm, dstbuf, acc):
        wid = lax.axis_index("s") * 2 + lax.axis_index("c")

        def zero_body(i, _):
            acc[pl.ds(i * LANES, LANES)] = jnp.zeros((LANES,), jnp.float32)
            return 0

        lax.fori_loop(0, NP // LANES, zero_body, 0)

        base = wid * EPT
        ones = jnp.ones((LANES,), jnp.float32)

        def chunk_body(ci, _):
            pltpu.sync_copy(dst_hbm.at[pl.ds(base + ci * CH, CH)], dstbuf)

            def in_body(j, _):
                idx = dstbuf[pl.ds(j * LANES, LANES)]
                plsc.addupdate_scatter(acc, [idx], ones)
                return 0

            lax.fori_loop(0, CH // LANES, in_body, 0)
            return 0

        lax.fori_loop(0, EPT // CH, chunk_body, 0)
        pltpu.sync_copy(acc, out_hbm.at[wid])

    return k(dst_pad)


def _sc_scalar_props(src_pad, dst_pad, xs, dinv, EP):
    EPT = EP // NW
    CH = 2048

    @functools.partial(
        pl.kernel,
        mesh=_mesh(),
        compiler_params=pltpu.CompilerParams(needs_layout_passes=False),
        out_type=[
            jax.ShapeDtypeStruct((NW, NP), jnp.float32),
            jax.ShapeDtypeStruct((NW, NP), jnp.float32),
        ],
        scratch_types=[
            pltpu.VMEM((CH,), jnp.int32),
            pltpu.VMEM((CH,), jnp.int32),
            pltpu.VMEM((NP,), jnp.float32),
            pltpu.VMEM((NP,), jnp.float32),
            pltpu.VMEM((NP,), jnp.float32),
            pltpu.VMEM((NP,), jnp.float32),
        ],
    )
    def k(src_hbm, dst_hbm, xs_hbm, dinv_hbm, outy_hbm, outw_hbm,
          srcbuf, dstbuf, xsl, dinvl, accy, accw):
        wid = lax.axis_index("s") * 2 + lax.axis_index("c")
        pltpu.sync_copy(xs_hbm, xsl)
        pltpu.sync_copy(dinv_hbm, dinvl)

        def zero_body(i, _):
            z = jnp.zeros((LANES,), jnp.float32)
            accy[pl.ds(i * LANES, LANES)] = z
            accw[pl.ds(i * LANES, LANES)] = z
            return 0

        lax.fori_loop(0, NP // LANES, zero_body, 0)

        base = wid * EPT

        def chunk_body(ci, _):
            pltpu.sync_copy(src_hbm.at[pl.ds(base + ci * CH, CH)], srcbuf)
            pltpu.sync_copy(dst_hbm.at[pl.ds(base + ci * CH, CH)], dstbuf)

            def in_body(j, _):
                sv = srcbuf[pl.ds(j * LANES, LANES)]
                dv = dstbuf[pl.ds(j * LANES, LANES)]
                xg = plsc.load_gather(xsl, [sv])
                plsc.addupdate_scatter(accy, [dv], xg)
                dg = plsc.load_gather(dinvl, [dv])
                plsc.addupdate_scatter(accw, [sv], dg)
                return 0

            lax.fori_loop(0, CH // LANES, in_body, 0)
            return 0

        lax.fori_loop(0, EPT // CH, chunk_body, 0)
        pltpu.sync_copy(accy, outy_hbm.at[wid])
        pltpu.sync_copy(accw, outw_hbm.at[wid])

    return k(src_pad, dst_pad, xs, dinv)


def _sc_row_prop(src2d, dst2d, zs, zrows, EP):
    ROWS = EP // 128
    RPT = ROWS // NW
    GROUPS = RPT // 16

    @functools.partial(
        pl.kernel,
        mesh=_mesh(),
        compiler_params=pltpu.CompilerParams(needs_layout_passes=False),
        out_type=jax.ShapeDtypeStruct((2, NP, 128), jnp.float32),
        scratch_types=[
            pltpu.VMEM((16, 128), jnp.int32),
            pltpu.VMEM((16, 128), jnp.int32),
            pltpu.VMEM((128, 128), jnp.float32),
            pltpu.VMEM((128, 128), jnp.float32),
            pltpu.VMEM((64, 128), jnp.float32),
            pltpu.VMEM_SHARED((NP, 128), jnp.float32),
            pltpu.SemaphoreType.DMA,
            pltpu.SemaphoreType.DMA,
        ],
    )
    def k(src_hbm, dst_hbm, zs_hbm, zrows_hbm, out_hbm,
          srcbuf, dstbuf, rb0, rb1, zbuf, acc, sem0, sem1):
        c = lax.axis_index("c")
        s = lax.axis_index("s")
        wid = s * 2 + c
        spt = NP // 16

        pltpu.sync_copy(zrows_hbm, zbuf)

        def zloop(t, _):
            pltpu.sync_copy(zbuf, acc.at[pl.ds(s * spt + t * 64, 64)])
            return 0

        lax.fori_loop(0, spt // 64, zloop, 0)
        plsc.subcore_barrier()

        base_row = wid * RPT
        bufs = (rb0, rb1)
        sems = (sem0, sem1)

        def grp(g, _):
            pltpu.sync_copy(src_hbm.at[pl.ds(base_row + g * 16, 16)], srcbuf)
            pltpu.sync_copy(dst_hbm.at[pl.ds(base_row + g * 16, 16)], dstbuf)
            h = pltpu.async_copy(zs_hbm.at[srcbuf.at[0]], bufs[0], sems[0])
            handles = [h]
            for j in range(16):
                if j < 15:
                    p = (j + 1) % 2
                    handles.append(pltpu.async_copy(
                        zs_hbm.at[srcbuf.at[j + 1]], bufs[p], sems[p]))
                handles[j].wait()
                pltpu.sync_copy(bufs[j % 2], acc.at[dstbuf.at[j]], add=True)
            return 0

        lax.fori_loop(0, GROUPS, grp, 0)
        plsc.subcore_barrier()
        pltpu.sync_copy(acc.at[pl.ds(s * spt, spt)],
                        out_hbm.at[c, pl.ds(s * spt, spt)])

    return k(src2d, dst2d, zs, zrows)


def _tc_prep(deg_parts, xp):
    def body(degp_ref, xp_ref, dinv_ref, xs_ref):
        deg = jnp.sum(degp_ref[...], axis=0, keepdims=True) + 1.0
        idx = lax.broadcasted_iota(jnp.int32, (1, NP), 1)
        dinv = jnp.where(idx < N, lax.rsqrt(deg), 0.0)
        dinv_ref[...] = dinv
        xs_ref[...] = dinv * xp_ref[...]

    return pl.pallas_call(
        body,
        out_shape=(jax.ShapeDtypeStruct((1, NP), jnp.float32),
                   jax.ShapeDtypeStruct((1, NP), jnp.float32)),
    )(deg_parts, xp)


def _tc_dense_mid(accy_parts, accw_parts, xs, dinv, W1, b1, W2):
    RB = 2048
    H1 = W1.shape[1]
    H2 = W2.shape[1]

    def body(ayp_ref, awp_ref, xs_ref, dinv_ref, W1_ref, b1_ref, W2_ref,
             zs_ref, w_ref):
        dinv = dinv_ref[0, :]
        ay = jnp.sum(ayp_ref[...], axis=0) + xs_ref[0, :]
        y1 = dinv * ay
        aw = jnp.sum(awp_ref[...], axis=0) + dinv
        w_ref[...] = (dinv * aw)[None, :]
        h1 = _leaky(y1[:, None] * W1_ref[...] + b1_ref[...][None, :])
        z = jnp.dot(h1, W2_ref[...], preferred_element_type=jnp.float32,
                    precision=lax.Precision.HIGHEST)
        zs_ref[...] = z * dinv[:, None]

    grid = (NP // RB,)
    return pl.pallas_call(
        body,
        grid=grid,
        in_specs=[
            pl.BlockSpec((NW, RB), lambda i: (0, i)),
            pl.BlockSpec((NW, RB), lambda i: (0, i)),
            pl.BlockSpec((1, RB), lambda i: (0, i)),
            pl.BlockSpec((1, RB), lambda i: (0, i)),
            pl.BlockSpec((1, H1), lambda i: (0, 0)),
            pl.BlockSpec((H1,), lambda i: (0,)),
            pl.BlockSpec((H1, H2), lambda i: (0, 0)),
        ],
        out_specs=(pl.BlockSpec((RB, H2), lambda i: (i, 0)),
                   pl.BlockSpec((1, RB), lambda i: (0, i))),
        out_shape=(jax.ShapeDtypeStruct((NP, H2), jnp.float32),
                   jax.ShapeDtypeStruct((1, NP), jnp.float32)),
    )(accy_parts, accw_parts, xs, dinv, W1, b1, W2)


def _tc_final(prop, zs, dinv, w, b2, W3, b3, Wm1, bm1, Wm2, bm2, Wm3, bm3):
    RC = 2048
    H2 = zs.shape[1]
    STEPS = NP // RC

    def body(p_ref, zs_ref, dinv_ref, w_ref, b2_ref, W3_ref, b3_ref,
             Wm1_ref, bm1_ref, Wm2_ref, bm2_ref, Wm3_ref, bm3_ref,
             out_ref, u_ref):
        i = pl.program_id(0)

        @pl.when(i == 0)
        def _():
            u_ref[...] = jnp.zeros_like(u_ref)

        dinv = dinv_ref[0, :][:, None]
        h2 = _leaky(dinv * (p_ref[0] + p_ref[1] + zs_ref[...])
                    + b2_ref[...][None, :])
        u_ref[...] += jnp.dot(w_ref[...], h2,
                              preferred_element_type=jnp.float32,
                              precision=lax.Precision.HIGHEST)

        @pl.when(i == STEPS - 1)
        def _():
            hp = lax.Precision.HIGHEST
            g = jnp.dot(u_ref[...] / N, W3_ref[...],
                        preferred_element_type=jnp.float32,
                        precision=hp) + b3_ref[...][None, :]
            m = _leaky(jnp.dot(g, Wm1_ref[...],
                               preferred_element_type=jnp.float32,
                               precision=hp) + bm1_ref[...][None, :])
            m = _leaky(jnp.dot(m, Wm2_ref[...],
                               preferred_element_type=jnp.float32,
                               precision=hp) + bm2_ref[...][None, :])
            o = jnp.dot(m, Wm3_ref[...], preferred_element_type=jnp.float32,
                        precision=hp) + bm3_ref[...][None, :]
            out_ref[...] = jax.nn.sigmoid(o)

    OD = W3.shape[1]
    M1 = Wm1.shape[1]
    M2 = Wm2.shape[1]
    MO = Wm3.shape[1]
    full2 = lambda d0, d1: pl.BlockSpec((d0, d1), lambda i: (0, 0))
    full1 = lambda d0: pl.BlockSpec((d0,), lambda i: (0,))
    return pl.pallas_call(
        body,
        grid=(STEPS,),
        in_specs=[
            pl.BlockSpec((2, RC, H2), lambda i: (0, i, 0)),
            pl.BlockSpec((RC, H2), lambda i: (i, 0)),
            pl.BlockSpec((1, RC), lambda i: (0, i)),
            pl.BlockSpec((1, RC), lambda i: (0, i)),
            full1(H2), full2(H2, OD), full1(OD),
            full2(OD, M1), full1(M1),
            full2(M1, M2), full1(M2),
            full2(M2, MO), full1(MO),
        ],
        out_specs=pl.BlockSpec((1, MO), lambda i: (0, 0)),
        out_shape=jax.ShapeDtypeStruct((1, MO), jnp.float32),
        scratch_shapes=[pltpu.VMEM((1, H2), jnp.float32)],
    )(prop, zs, dinv, w, b2, W3, b3, Wm1, bm1, Wm2, bm2, Wm3, bm3)


def kernel(x, edge_index, W1, b1, W2, b2, W3, b3,
           Wm1, bm1, Wm2, bm2, Wm3, bm3):
    E0 = edge_index.shape[1]
    EP = ((E0 + 65535) // 65536) * 65536

    pad = EP - E0
    padv = jnp.full((pad,), NP - 1, dtype=edge_index.dtype)
    src = jnp.concatenate([edge_index[0], padv])
    dst = jnp.concatenate([edge_index[1], padv])
    xp = jnp.pad(x[:, 0], (0, NP - N)).reshape(1, NP)
    zrows = jnp.zeros((64, 128), jnp.float32)

    deg_parts = _sc_degree(dst, EP)
    dinv, xs = _tc_prep(deg_parts, xp)
    accy, accw = _sc_scalar_props(src, dst, xs.reshape(NP), dinv.reshape(NP),
                                  EP)
    zs, w = _tc_dense_mid(accy, accw, xs, dinv, W1, b1, W2)
    prop = _sc_row_prop(src.reshape(EP // 128, 128),
                        dst.reshape(EP // 128, 128), zs, zrows, EP)
    out = _tc_final(prop, zs, dinv, w, b2, W3, b3,
                    Wm1, bm1, Wm2, bm2, Wm3, bm3)
    return out.reshape(bm3.shape)


# speedup vs baseline: 23.1902x; 1.1518x over previous
import functools

import jax
import jax.numpy as jnp
from jax import lax
from jax.experimental import pallas as pl
from jax.experimental.pallas import tpu as pltpu
from jax.experimental.pallas import tpu_sc as plsc

N = 10000
NP = 10240
LANES = 16
NW = 32


def _leaky(v):
    return jnp.where(v > 0, v, 0.1 * v)


def _mesh():
    return plsc.VectorSubcoreMesh(core_axis_name="c", subcore_axis_name="s")


def _sc_degree(dst_pad, EP):
    EPT = EP // NW
    CH = 2048

    @functools.partial(
        pl.kernel,
        mesh=_mesh(),
        compiler_params=pltpu.CompilerParams(needs_layout_passes=False),
        out_type=jax.ShapeDtypeStruct((NW, NP), jnp.float32),
        scratch_types=[
            pltpu.VMEM((CH,), jnp.int32),
            pltpu.VMEM((NP,), jnp.float32),
        ],
    )
    def k(dst_hbm, out_hbm, dstbuf, acc):
        wid = lax.axis_index("s") * 2 + lax.axis_index("c")

        def zero_body(i, _):
            acc[pl.ds(i * LANES, LANES)] = jnp.zeros((LANES,), jnp.float32)
            return 0

        lax.fori_loop(0, NP // LANES, zero_body, 0)

        base = wid * EPT
        ones = jnp.ones((LANES,), jnp.float32)

        def chunk_body(ci, _):
            pltpu.sync_copy(dst_hbm.at[pl.ds(base + ci * CH, CH)], dstbuf)

            def in_body(j, _):
                idx = dstbuf[pl.ds(j * LANES, LANES)]
                plsc.addupdate_scatter(acc, [idx], ones)
                return 0

            lax.fori_loop(0, CH // LANES, in_body, 0)
            return 0

        lax.fori_loop(0, EPT // CH, chunk_body, 0)
        pltpu.sync_copy(acc, out_hbm.at[wid])

    return k(dst_pad)


def _sc_scalar_props(src_pad, dst_pad, xs, dinv, EP):
    EPT = EP // NW
    CH = 2048

    @functools.partial(
        pl.kernel,
        mesh=_mesh(),
        compiler_params=pltpu.CompilerParams(needs_layout_passes=False),
        out_type=[
            jax.ShapeDtypeStruct((NW, NP), jnp.float32),
            jax.ShapeDtypeStruct((NW, NP), jnp.float32),
        ],
        scratch_types=[
            pltpu.VMEM((CH,), jnp.int32),
            pltpu.VMEM((CH,), jnp.int32),
            pltpu.VMEM((NP,), jnp.float32),
            pltpu.VMEM((NP,), jnp.float32),
            pltpu.VMEM((NP,), jnp.float32),
            pltpu.VMEM((NP,), jnp.float32),
        ],
    )
    def k(src_hbm, dst_hbm, xs_hbm, dinv_hbm, outy_hbm, outw_hbm,
          srcbuf, dstbuf, xsl, dinvl, accy, accw):
        wid = lax.axis_index("s") * 2 + lax.axis_index("c")
        pltpu.sync_copy(xs_hbm, xsl)
        pltpu.sync_copy(dinv_hbm, dinvl)

        def zero_body(i, _):
            z = jnp.zeros((LANES,), jnp.float32)
            accy[pl.ds(i * LANES, LANES)] = z
            accw[pl.ds(i * LANES, LANES)] = z
            return 0

        lax.fori_loop(0, NP // LANES, zero_body, 0)

        base = wid * EPT

        def chunk_body(ci, _):
            pltpu.sync_copy(src_hbm.at[pl.ds(base + ci * CH, CH)], srcbuf)
            pltpu.sync_copy(dst_hbm.at[pl.ds(base + ci * CH, CH)], dstbuf)

            def in_body(j, _):
                sv = srcbuf[pl.ds(j * LANES, LANES)]
                dv = dstbuf[pl.ds(j * LANES, LANES)]
                xg = plsc.load_gather(xsl, [sv])
                plsc.addupdate_scatter(accy, [dv], xg)
                dg = plsc.load_gather(dinvl, [dv])
                plsc.addupdate_scatter(accw, [sv], dg)
                return 0

            lax.fori_loop(0, CH // LANES, in_body, 0)
            return 0

        lax.fori_loop(0, EPT // CH, chunk_body, 0)
        pltpu.sync_copy(accy, outy_hbm.at[wid])
        pltpu.sync_copy(accw, outw_hbm.at[wid])

    return k(src_pad, dst_pad, xs, dinv)


def _sc_row_prop(src2d, dst2d, zs, zrows, EP):
    ROWS = EP // 128
    RF = (ROWS * 85 // (100 * 16 * 16)) * 16
    RS = ROWS // 16 - RF
    GF, GS = RF // 16, RS // 16

    @functools.partial(
        pl.kernel,
        mesh=_mesh(),
        compiler_params=pltpu.CompilerParams(needs_layout_passes=False),
        out_type=jax.ShapeDtypeStruct((2, NP, 128), jnp.float32),
        scratch_types=[
            pltpu.VMEM((16, 128), jnp.int32),
            pltpu.VMEM((16, 128), jnp.int32),
            pltpu.VMEM((128, 128), jnp.float32),
            pltpu.VMEM((128, 128), jnp.float32),
            pltpu.VMEM_SHARED((NP, 128), jnp.float32),
            pltpu.SemaphoreType.DMA,
            pltpu.SemaphoreType.DMA,
        ],
    )
    def k(src_hbm, dst_hbm, zs_hbm, zrows_hbm, out_hbm,
          srcbuf, dstbuf, rb0, rb1, acc, sem0, sem1):
        c = lax.axis_index("c")
        s = lax.axis_index("s")
        spt = NP // 16

        pltpu.sync_copy(zrows_hbm, acc.at[pl.ds(s * spt, spt)])
        plsc.subcore_barrier()

        base_row = jnp.where(c == 0, s * RF, 16 * RF + s * RS)
        n_groups = jnp.where(c == 0, GF, GS)
        bufs = (rb0, rb1)
        sems = (sem0, sem1)

        def grp(g, _):
            pltpu.sync_copy(src_hbm.at[pl.ds(base_row + g * 16, 16)], srcbuf)
            pltpu.sync_copy(dst_hbm.at[pl.ds(base_row + g * 16, 16)], dstbuf)
            h = pltpu.async_copy(zs_hbm.at[srcbuf.at[0]], bufs[0], sems[0])
            handles = [h]
            for j in range(16):
                if j < 15:
                    p = (j + 1) % 2
                    handles.append(pltpu.async_copy(
                        zs_hbm.at[srcbuf.at[j + 1]], bufs[p], sems[p]))
                handles[j].wait()
                pltpu.sync_copy(bufs[j % 2], acc.at[dstbuf.at[j]], add=True)
            return 0

        lax.fori_loop(0, n_groups, grp, 0)
        plsc.subcore_barrier()
        pltpu.sync_copy(acc.at[pl.ds(s * spt, spt)],
                        out_hbm.at[c, pl.ds(s * spt, spt)])

    return k(src2d, dst2d, zs, zrows)


def _tc_prep(deg_parts, xp):
    def body(degp_ref, xp_ref, dinv_ref, xs_ref):
        deg = jnp.sum(degp_ref[...], axis=0, keepdims=True) + 1.0
        idx = lax.broadcasted_iota(jnp.int32, (1, NP), 1)
        dinv = jnp.where(idx < N, lax.rsqrt(deg), 0.0)
        dinv_ref[...] = dinv
        xs_ref[...] = dinv * xp_ref[...]

    return pl.pallas_call(
        body,
        out_shape=(jax.ShapeDtypeStruct((1, NP), jnp.float32),
                   jax.ShapeDtypeStruct((1, NP), jnp.float32)),
    )(deg_parts, xp)


def _tc_dense_mid(accy_parts, accw_parts, xs, dinv, W1, b1, W2):
    RB = 2048
    H1 = W1.shape[1]
    H2 = W2.shape[1]

    def body(ayp_ref, awp_ref, xs_ref, dinv_ref, W1_ref, b1_ref, W2_ref,
             zs_ref, w_ref):
        dinv = dinv_ref[0, :]
        ay = jnp.sum(ayp_ref[...], axis=0) + xs_ref[0, :]
        y1 = dinv * ay
        aw = jnp.sum(awp_ref[...], axis=0) + dinv
        w_ref[...] = (dinv * aw)[None, :]
        h1 = _leaky(y1[:, None] * W1_ref[...] + b1_ref[...][None, :])
        z = jnp.dot(h1, W2_ref[...], preferred_element_type=jnp.float32,
                    precision=lax.Precision.HIGHEST)
        zs_ref[...] = z * dinv[:, None]

    grid = (NP // RB,)
    return pl.pallas_call(
        body,
        grid=grid,
        in_specs=[
            pl.BlockSpec((NW, RB), lambda i: (0, i)),
            pl.BlockSpec((NW, RB), lambda i: (0, i)),
            pl.BlockSpec((1, RB), lambda i: (0, i)),
            pl.BlockSpec((1, RB), lambda i: (0, i)),
            pl.BlockSpec((1, H1), lambda i: (0, 0)),
            pl.BlockSpec((H1,), lambda i: (0,)),
            pl.BlockSpec((H1, H2), lambda i: (0, 0)),
        ],
        out_specs=(pl.BlockSpec((RB, H2), lambda i: (i, 0)),
                   pl.BlockSpec((1, RB), lambda i: (0, i))),
        out_shape=(jax.ShapeDtypeStruct((NP, H2), jnp.float32),
                   jax.ShapeDtypeStruct((1, NP), jnp.float32)),
    )(accy_parts, accw_parts, xs, dinv, W1, b1, W2)


def _tc_final(prop, zs, dinv, w, b2, W3, b3,
              Wm1, bm1, Wm2, bm2, Wm3, bm3):
    RC = 2048
    H2 = W3.shape[0]
    STEPS = NP // RC

    def body(p_ref, zs_ref, dinv_ref, w_ref, b2_ref, W3_ref, b3_ref,
             Wm1_ref, bm1_ref, Wm2_ref, bm2_ref, Wm3_ref, bm3_ref,
             out_ref, u_ref):
        i = pl.program_id(0)

        @pl.when(i == 0)
        def _():
            u_ref[...] = jnp.zeros_like(u_ref)

        dinv = dinv_ref[0, :][:, None]
        hp = lax.Precision.HIGHEST
        h2 = _leaky(dinv * (p_ref[0] + p_ref[1] + zs_ref[...])
                    + b2_ref[...][None, :])
        u_ref[...] += jnp.dot(w_ref[...], h2,
                              preferred_element_type=jnp.float32,
                              precision=hp)

        @pl.when(i == STEPS - 1)
        def _():
            g = jnp.dot(u_ref[...] / N, W3_ref[...],
                        preferred_element_type=jnp.float32,
                        precision=hp) + b3_ref[...][None, :]
            m = _leaky(jnp.dot(g, Wm1_ref[...],
                               preferred_element_type=jnp.float32,
                               precision=hp) + bm1_ref[...][None, :])
            m = _leaky(jnp.dot(m, Wm2_ref[...],
                               preferred_element_type=jnp.float32,
                               precision=hp) + bm2_ref[...][None, :])
            o = jnp.dot(m, Wm3_ref[...], preferred_element_type=jnp.float32,
                        precision=hp) + bm3_ref[...][None, :]
            out_ref[...] = jax.nn.sigmoid(o)

    OD = W3.shape[1]
    M1 = Wm1.shape[1]
    M2 = Wm2.shape[1]
    MO = Wm3.shape[1]
    full2 = lambda d0, d1: pl.BlockSpec((d0, d1), lambda i: (0, 0))
    full1 = lambda d0: pl.BlockSpec((d0,), lambda i: (0,))
    return pl.pallas_call(
        body,
        grid=(STEPS,),
        in_specs=[
            pl.BlockSpec((2, RC, H2), lambda i: (0, i, 0)),
            pl.BlockSpec((RC, H2), lambda i: (i, 0)),
            pl.BlockSpec((1, RC), lambda i: (0, i)),
            pl.BlockSpec((1, RC), lambda i: (0, i)),
            full1(H2), full2(H2, OD), full1(OD),
            full2(OD, M1), full1(M1),
            full2(M1, M2), full1(M2),
            full2(M2, MO), full1(MO),
        ],
        out_specs=pl.BlockSpec((1, MO), lambda i: (0, 0)),
        out_shape=jax.ShapeDtypeStruct((1, MO), jnp.float32),
        scratch_shapes=[pltpu.VMEM((1, H2), jnp.float32)],
    )(prop, zs, dinv, w, b2, W3, b3, Wm1, bm1, Wm2, bm2, Wm3, bm3)


def kernel(x, edge_index, W1, b1, W2, b2, W3, b3,
           Wm1, bm1, Wm2, bm2, Wm3, bm3):
    E0 = edge_index.shape[1]
    EP = ((E0 + 65535) // 65536) * 65536

    pad = EP - E0
    padv = jnp.full((pad,), NP - 1, dtype=edge_index.dtype)
    src = jnp.concatenate([edge_index[0], padv])
    dst = jnp.concatenate([edge_index[1], padv])
    xp = jnp.pad(x[:, 0], (0, NP - N)).reshape(1, NP)
    zrows = jnp.zeros((NP // 16, 128), jnp.float32)

    deg_parts = _sc_degree(dst, EP)
    dinv, xs = _tc_prep(deg_parts, xp)
    accy, accw = _sc_scalar_props(src, dst, xs.reshape(NP), dinv.reshape(NP),
                                  EP)
    zs, w = _tc_dense_mid(accy, accw, xs, dinv, W1, b1, W2)
    prop = _sc_row_prop(src.reshape(EP // 128, 128),
                        dst.reshape(EP // 128, 128), zs, zrows, EP)
    out = _tc_final(prop, zs, dinv, w, b2, W3, b3,
                    Wm1, bm1, Wm2, bm2, Wm3, bm3)
    return out.reshape(bm3.shape)
